# baseline (speedup 1.0000x reference)
import functools

import jax
import jax.numpy as jnp
from jax import lax
from jax.experimental import pallas as pl
from jax.experimental.pallas import tpu as pltpu
from jax.experimental.pallas import tpu_sc as plsc

_NC = 2
_NS = 16
_NW = _NC * _NS
_CHUNK = 128
_K = 2
_SLOTS = 4
_LANES = 128


@functools.lru_cache(maxsize=None)
def _make_transpose(v, d):
    tcols = v // _LANES
    tail = v - tcols * _LANES
    base_cols = tcols // _NW
    extra = tcols - base_cols * _NW
    mesh = plsc.VectorSubcoreMesh(core_axis_name="c", subcore_axis_name="s")

    def body(wt_hbm, tail_hbm, out_hbm, blk, obuf, tbuf, gsem, osem):
        wid = lax.axis_index("s") * _NC + lax.axis_index("c")

        if tail:
            @pl.when(wid == 0)
            def _():
                pltpu.sync_copy(tail_hbm, tbuf)
                pltpu.sync_copy(tbuf, out_hbm.at[pl.ds(tcols * _LANES * d, tail * d)])

        iota = lax.iota(jnp.int32, 16)

        def do_one(j):
            pltpu.async_copy(
                wt_hbm.at[pl.ds(0, d), pl.ds(j * _LANES, _LANES)], blk, gsem
            ).wait()

            def trans_l(l, carry):
                colv = jnp.full((16,), 0, jnp.int32) + l
                for c in range(d // 16):
                    vv = plsc.load_gather(blk, [c * 16 + iota, colv])
                    obuf[pl.ds(l * d + c * 16, 16)] = vv
                return carry

            lax.fori_loop(0, _LANES, trans_l, 0)
            pltpu.async_copy(obuf, out_hbm.at[pl.ds(j * _LANES * d, _LANES * d)],
                             osem).wait()

        def step(i, carry):
            do_one(wid + _NW * i)
            return carry

        lax.fori_loop(0, base_cols, step, 0)
        if extra:
            @pl.when(wid < extra)
            def _():
                do_one(wid + _NW * base_cols)

    return pl.kernel(
        body,
        out_type=jax.ShapeDtypeStruct((v * d,), jnp.float32),
        mesh=mesh,
        scratch_types=[
            pltpu.VMEM((d, _LANES), jnp.float32),
            pltpu.VMEM((_LANES * d,), jnp.float32),
            pltpu.VMEM((max(tail, 1) * d,), jnp.float32),
            pltpu.SemaphoreType.DMA,
            pltpu.SemaphoreType.DMA,
        ],
        compiler_params=pltpu.CompilerParams(
            use_tc_tiling_on_sc=True, needs_layout_passes=False),
    )


@functools.lru_cache(maxsize=None)
def _make_gather(n, d, nchunk):
    per_w = n // _NW
    ngroups = nchunk // _K
    assert nchunk % _K == 0 and (ngroups - 4) % 4 == 0 and ngroups >= 8
    mesh = plsc.VectorSubcoreMesh(core_axis_name="c", subcore_axis_name="s")

    def body(idx_hbm, table_hbm, out_hbm, idx_v, rows_v,
             g0, g1, g2, g3, o0, o1, o2, o3):
        gsems = (g0, g1, g2, g3)
        osems = (o0, o1, o2, o3)
        wid = lax.axis_index("s") * _NC + lax.axis_index("c")
        pltpu.sync_copy(idx_hbm.at[wid], idx_v)
        base = wid * per_w

        def g_desc(g, s, b):
            c = g * _K + b
            return pltpu.make_async_copy(
                table_hbm.at[idx_v.at[c]], rows_v.at[s * _K + b], gsems[s])

        def o_desc(g, s, b):
            c = g * _K + b
            return pltpu.make_async_copy(
                rows_v.at[s * _K + b],
                out_hbm.at[pl.ds(base + c * _CHUNK, _CHUNK), pl.ds(0, d)],
                osems[s])

        def gfire(g, s):
            for b in range(_K):
                g_desc(g, s, b).start()

        def gdrain(g, s):
            for b in range(_K):
                g_desc(g, s, b).wait()

        def ofire(g, s):
            for b in range(_K):
                o_desc(g, s, b).start()

        def odrain(g, s):
            for b in range(_K):
                o_desc(g, s, b).wait()

        def part(g, s):
            odrain(g - 2, (s + 2) % _SLOTS)
            gfire(g + 2, (s + 2) % _SLOTS)
            gdrain(g, s)
            ofire(g, s)

        gfire(0, 0)
        gfire(1, 1)
        gfire(2, 2)
        gdrain(0, 0)
        ofire(0, 0)
        gfire(3, 3)
        gdrain(1, 1)
        ofire(1, 1)

        def step(i, carry):
            gbase = 4 * i + 2
            for q in range(4):
                part(gbase + q, (2 + q) % _SLOTS)
            return carry

        lax.fori_loop(0, (ngroups - 4) // 4, step, 0)

        ge = ngroups - 2
        odrain(ge - 2, 0)
        gdrain(ge, 2)
        ofire(ge, 2)
        odrain(ge - 1, 1)
        gdrain(ge + 1, 3)
        ofire(ge + 1, 3)
        odrain(ge, 2)
        odrain(ge + 1, 3)

    return pl.kernel(
        body,
        out_type=jax.ShapeDtypeStruct((n, 2 * d), jnp.float32),
        mesh=mesh,
        scratch_types=(
            [pltpu.VMEM((nchunk, _CHUNK), jnp.int32),
             pltpu.VMEM((_SLOTS * _K, _CHUNK, d), jnp.float32)]
            + [pltpu.SemaphoreType.DMA] * 8
        ),
        compiler_params=pltpu.CompilerParams(use_tc_tiling_on_sc=False),
    )


def kernel(input_ids, W_emb):
    b, l = input_ids.shape
    v, d = W_emb.shape
    n = b * l
    nchunk = n // (_NW * _CHUNK)
    idx = input_ids.reshape(_NW, nchunk, _CHUNK).astype(jnp.int32)

    tcols = v // _LANES
    tail = W_emb[tcols * _LANES:].reshape(-1)
    w_lin = _make_transpose(v, d)(W_emb.T, tail)

    out = _make_gather(n, d, nchunk)(idx, w_lin.reshape(v, d))
    return out[:, :d].reshape(b, l, d)


# speedup vs baseline: 1.1917x; 1.1917x over previous
import functools

import jax
import jax.numpy as jnp
from jax import lax
from jax.experimental import pallas as pl
from jax.experimental.pallas import tpu as pltpu
from jax.experimental.pallas import tpu_sc as plsc

_NC = 2
_NS = 16
_NW = _NC * _NS
_CHUNK = 128
_K = 2
_SLOTS = 4
_LANES = 128


@functools.lru_cache(maxsize=None)
def _make_transpose(v, d):
    tcols = v // _LANES
    tail = v - tcols * _LANES
    nparts = -(-tcols // _NW)
    assert nparts >= 4
    mesh = plsc.VectorSubcoreMesh(core_axis_name="c", subcore_axis_name="s")

    def body(wt_hbm, tail_hbm, out_hbm, blk0, blk1, obuf0, obuf1, tbuf,
             g0, g1, o0, o1):
        blks = (blk0, blk1)
        obufs = (obuf0, obuf1)
        gsems = (g0, g1)
        osems = (o0, o1)
        wid = lax.axis_index("s") * _NC + lax.axis_index("c")

        if tail:
            @pl.when(wid == 0)
            def _():
                pltpu.sync_copy(tail_hbm, tbuf)
                pltpu.sync_copy(tbuf, out_hbm.at[pl.ds(tcols * _LANES * d, tail * d)])

        iota = lax.iota(jnp.int32, 16)

        def jcol(t):
            return jnp.minimum(wid + _NW * t, tcols - 1)

        def s_desc(t, p):
            j = jcol(t)
            return pltpu.make_async_copy(
                wt_hbm.at[pl.ds(0, d), pl.ds(j * _LANES, _LANES)],
                blks[p], gsems[p])

        def o_desc(t, p):
            j = jcol(t)
            return pltpu.make_async_copy(
                obufs[p], out_hbm.at[pl.ds(j * _LANES * d, _LANES * d)],
                osems[p])

        def transpose_block(p):
            bp = blks[p]
            op = obufs[p]

            def tl(lb, carry):
                basev = jnp.full((16,), 0, jnp.int32) + lb * 8
                for li in range(8):
                    colv = basev + li
                    for c in range(d // 16):
                        vv = plsc.load_gather(bp, [c * 16 + iota, colv])
                        op[pl.ds((lb * 8 + li) * d + c * 16, 16)] = vv
                return carry

            lax.fori_loop(0, _LANES // 8, tl, 0)

        def part(t, p, fire_next=True, wait_out=True):
            s_desc(t, p).wait()
            if fire_next:
                s_desc(t + 1, 1 - p).start()
            if wait_out:
                o_desc(t - 2, p).wait()
            transpose_block(p)
            o_desc(t, p).start()

        s_desc(0, 0).start()
        part(0, 0, wait_out=False)
        part(1, 1, wait_out=False)

        def step(i, carry):
            t = 2 + 2 * i
            part(t, 0)
            part(t + 1, 1)
            return carry

        lax.fori_loop(0, (nparts - 3) // 2, step, 0)
        part(nparts - 1, (nparts - 1) % 2, fire_next=False)
        o_desc(nparts - 2, (nparts - 2) % 2).wait()
        o_desc(nparts - 1, (nparts - 1) % 2).wait()

    return pl.kernel(
        body,
        out_type=jax.ShapeDtypeStruct((v * d,), jnp.float32),
        mesh=mesh,
        scratch_types=[
            pltpu.VMEM((d, _LANES), jnp.float32),
            pltpu.VMEM((d, _LANES), jnp.float32),
            pltpu.VMEM((_LANES * d,), jnp.float32),
            pltpu.VMEM((_LANES * d,), jnp.float32),
            pltpu.VMEM((max(tail, 1) * d,), jnp.float32),
            pltpu.SemaphoreType.DMA,
            pltpu.SemaphoreType.DMA,
            pltpu.SemaphoreType.DMA,
            pltpu.SemaphoreType.DMA,
        ],
        compiler_params=pltpu.CompilerParams(
            use_tc_tiling_on_sc=True, needs_layout_passes=False),
    )


@functools.lru_cache(maxsize=None)
def _make_gather(n, d, nchunk):
    per_w = n // _NW
    ngroups = nchunk // _K
    assert nchunk % _K == 0 and (ngroups - 4) % 4 == 0 and ngroups >= 8
    mesh = plsc.VectorSubcoreMesh(core_axis_name="c", subcore_axis_name="s")

    def body(idx_hbm, table_hbm, out_hbm, idx_v, rows_v,
             g0, g1, g2, g3, o0, o1, o2, o3):
        gsems = (g0, g1, g2, g3)
        osems = (o0, o1, o2, o3)
        wid = lax.axis_index("s") * _NC + lax.axis_index("c")
        pltpu.sync_copy(idx_hbm.at[wid], idx_v)
        base = wid * per_w

        def g_desc(g, s, b):
            c = g * _K + b
            return pltpu.make_async_copy(
                table_hbm.at[idx_v.at[c]], rows_v.at[s * _K + b], gsems[s])

        def o_desc(g, s, b):
            c = g * _K + b
            return pltpu.make_async_copy(
                rows_v.at[s * _K + b],
                out_hbm.at[pl.ds(base + c * _CHUNK, _CHUNK), pl.ds(0, d)],
                osems[s])

        def gfire(g, s):
            for b in range(_K):
                g_desc(g, s, b).start()

        def gdrain(g, s):
            for b in range(_K):
                g_desc(g, s, b).wait()

        def ofire(g, s):
            for b in range(_K):
                o_desc(g, s, b).start()

        def odrain(g, s):
            for b in range(_K):
                o_desc(g, s, b).wait()

        def part(g, s):
            odrain(g - 2, (s + 2) % _SLOTS)
            gfire(g + 2, (s + 2) % _SLOTS)
            gdrain(g, s)
            ofire(g, s)

        gfire(0, 0)
        gfire(1, 1)
        gfire(2, 2)
        gdrain(0, 0)
        ofire(0, 0)
        gfire(3, 3)
        gdrain(1, 1)
        ofire(1, 1)

        def step(i, carry):
            gbase = 4 * i + 2
            for q in range(4):
                part(gbase + q, (2 + q) % _SLOTS)
            return carry

        lax.fori_loop(0, (ngroups - 4) // 4, step, 0)

        ge = ngroups - 2
        odrain(ge - 2, 0)
        gdrain(ge, 2)
        ofire(ge, 2)
        odrain(ge - 1, 1)
        gdrain(ge + 1, 3)
        ofire(ge + 1, 3)
        odrain(ge, 2)
        odrain(ge + 1, 3)

    return pl.kernel(
        body,
        out_type=jax.ShapeDtypeStruct((n, 2 * d), jnp.float32),
        mesh=mesh,
        scratch_types=(
            [pltpu.VMEM((nchunk, _CHUNK), jnp.int32),
             pltpu.VMEM((_SLOTS * _K, _CHUNK, d), jnp.float32)]
            + [pltpu.SemaphoreType.DMA] * 8
        ),
        compiler_params=pltpu.CompilerParams(use_tc_tiling_on_sc=False),
    )


def kernel(input_ids, W_emb):
    b, l = input_ids.shape
    v, d = W_emb.shape
    n = b * l
    nchunk = n // (_NW * _CHUNK)
    idx = input_ids.reshape(_NW, nchunk, _CHUNK).astype(jnp.int32)

    tcols = v // _LANES
    tail = W_emb[tcols * _LANES:].reshape(-1)
    w_lin = _make_transpose(v, d)(W_emb.T, tail)

    out = _make_gather(n, d, nchunk)(idx, w_lin.reshape(v, d))
    return out[:, :d].reshape(b, l, d)


# speedup vs baseline: 2.3420x; 1.9654x over previous
import functools

import jax
import jax.numpy as jnp
from jax import lax
from jax.experimental import pallas as pl
from jax.experimental.pallas import tpu as pltpu
from jax.experimental.pallas import tpu_sc as plsc

_NC = 2
_NS = 16
_NW = _NC * _NS
_CHUNK = 128
_K = 2
_SLOTS = 4
_LANES = 128


@functools.lru_cache(maxsize=None)
def _make_gather(n, d, nchunk):
    per_w = n // _NW
    ngroups = nchunk // _K
    assert nchunk % _K == 0 and (ngroups - 4) % 4 == 0 and ngroups >= 8
    mesh = plsc.VectorSubcoreMesh(core_axis_name="c", subcore_axis_name="s")

    def body(idx_hbm, table_hbm, out_hbm, idx_v, rows_v,
             g0, g1, g2, g3, o0, o1, o2, o3):
        gsems = (g0, g1, g2, g3)
        osems = (o0, o1, o2, o3)
        wid = lax.axis_index("s") * _NC + lax.axis_index("c")
        pltpu.sync_copy(idx_hbm.at[wid], idx_v)
        base = wid * per_w

        def g_desc(g, s, b):
            c = g * _K + b
            return pltpu.make_async_copy(
                table_hbm.at[idx_v.at[c]], rows_v.at[s * _K + b], gsems[s])

        def o_desc(g, s, b):
            c = g * _K + b
            return pltpu.make_async_copy(
                rows_v.at[s * _K + b],
                out_hbm.at[pl.ds(base + c * _CHUNK, _CHUNK), pl.ds(0, d)],
                osems[s])

        def gfire(g, s):
            for b in range(_K):
                g_desc(g, s, b).start()

        def gdrain(g, s):
            for b in range(_K):
                g_desc(g, s, b).wait()

        def ofire(g, s):
            for b in range(_K):
                o_desc(g, s, b).start()

        def odrain(g, s):
            for b in range(_K):
                o_desc(g, s, b).wait()

        def part(g, s):
            odrain(g - 2, (s + 2) % _SLOTS)
            gfire(g + 2, (s + 2) % _SLOTS)
            gdrain(g, s)
            ofire(g, s)

        gfire(0, 0)
        gfire(1, 1)
        gfire(2, 2)
        gdrain(0, 0)
        ofire(0, 0)
        gfire(3, 3)
        gdrain(1, 1)
        ofire(1, 1)

        def step(i, carry):
            gbase = 4 * i + 2
            for q in range(4):
                part(gbase + q, (2 + q) % _SLOTS)
            return carry

        lax.fori_loop(0, (ngroups - 4) // 4, step, 0)

        ge = ngroups - 2
        odrain(ge - 2, 0)
        gdrain(ge, 2)
        ofire(ge, 2)
        odrain(ge - 1, 1)
        gdrain(ge + 1, 3)
        ofire(ge + 1, 3)
        odrain(ge, 2)
        odrain(ge + 1, 3)

    return pl.kernel(
        body,
        out_type=jax.ShapeDtypeStruct((n, 2 * d), jnp.float32),
        mesh=mesh,
        scratch_types=(
            [pltpu.VMEM((nchunk, _CHUNK), jnp.int32),
             pltpu.VMEM((_SLOTS * _K, _CHUNK, d), jnp.float32)]
            + [pltpu.SemaphoreType.DMA] * 8
        ),
        compiler_params=pltpu.CompilerParams(use_tc_tiling_on_sc=False),
    )


def kernel(input_ids, W_emb):
    b, l = input_ids.shape
    v, d = W_emb.shape
    n = b * l
    nchunk = n // (_NW * _CHUNK)
    idx = input_ids.reshape(_NW, nchunk, _CHUNK).astype(jnp.int32)
    out = _make_gather(n, d, nchunk)(idx, W_emb)
    return out[:, :d].reshape(b, l, d)
